# baseline (speedup 1.0000x reference)
import dataclasses

import jax
import jax.numpy as jnp
from jax.experimental import pallas as pl
from jax.experimental.pallas import tpu as pltpu
from jax.experimental.pallas import tpu_sc as plsc

_NUM_CLASSES = 64
_BLOCK_B = 512
_SPLIT = 0
_B = 16384
_E = 2016
_N_WORKERS = 32
_T = 16
_N_CHUNKS = _E // 16


def _tc_vote_kernel(x_ref, perms_ref, out_ref, m_ref, base_ref):
    @pl.when(pl.program_id(0) == 0)
    def _build_votes():
        c_iota = jax.lax.broadcasted_iota(
            jnp.int32, (perms_ref.shape[0], _NUM_CLASSES), 1
        )
        lmat = (perms_ref[:, 0:1] == c_iota).astype(jnp.float32)
        rmat = (perms_ref[:, 1:2] == c_iota).astype(jnp.float32)
        m_ref[...] = (rmat - lmat).astype(jnp.bfloat16)
        base_ref[...] = jnp.broadcast_to(
            jnp.sum(lmat, axis=0, keepdims=True), base_ref.shape
        )

    v = (x_ref[...] > 0.5).astype(jnp.bfloat16)
    counts = jax.lax.dot_general(
        v, m_ref[...], (((1,), (0,)), ((), ())), preferred_element_type=jnp.float32
    ) + base_ref[0:1, :]

    out_iota = jax.lax.broadcasted_iota(jnp.int32, counts.shape, 1)
    score = counts * float(_NUM_CLASSES) + (_NUM_CLASSES - 1 - out_iota).astype(
        jnp.float32
    )
    best = jnp.max(score, axis=1, keepdims=True)
    out_ref[...] = (score == best).astype(jnp.int32)


def _tc_part(x, perms, n_rows):
    return pl.pallas_call(
        _tc_vote_kernel,
        grid=(n_rows // _BLOCK_B,),
        in_specs=[
            pl.BlockSpec((_BLOCK_B, _E), lambda i: (i, 0)),
            pl.BlockSpec((_E, 2), lambda i: (0, 0)),
        ],
        out_specs=pl.BlockSpec((_BLOCK_B, _NUM_CLASSES), lambda i: (i, 0)),
        out_shape=jax.ShapeDtypeStruct((n_rows, _NUM_CLASSES), jnp.int32),
        scratch_shapes=[
            pltpu.VMEM((_E, _NUM_CLASSES), jnp.bfloat16),
            pltpu.VMEM((8, _NUM_CLASSES), jnp.float32),
        ],
    )(x, perms)


def _sc_part(x, left, right, row_offset, n_rows):
    n_tiles_per_worker = n_rows // (_N_WORKERS * _T)
    mesh = plsc.VectorSubcoreMesh(core_axis_name="c", subcore_axis_name="s")
    cp = pltpu.CompilerParams()
    if "needs_layout_passes" in pltpu.CompilerParams.__dataclass_fields__:
        cp = dataclasses.replace(cp, needs_layout_passes=False)

    @pl.kernel(
        out_type=jax.ShapeDtypeStruct((n_rows, _NUM_CLASSES), jnp.int32),
        mesh=mesh,
        compiler_params=cp,
        scratch_types=[
            pltpu.VMEM((_E,), jnp.int32),
            pltpu.VMEM((_E,), jnp.int32),
            pltpu.VMEM((_T * _NUM_CLASSES,), jnp.int32),
            pltpu.SemaphoreType.DMA,
        ],
    )
    def sc_kernel(x_hbm, l_hbm, r_hbm, o_hbm, l_vmem, r_vmem, counts, sem):
        pltpu.async_copy(l_hbm, l_vmem, sem).wait()
        pltpu.async_copy(r_hbm, r_vmem, sem).wait()

        lane = jax.lax.iota(jnp.int32, 16)
        ones = jnp.full((16,), 1, jnp.int32)
        zeros = jnp.zeros((16,), jnp.int32)

        def body(x_vmem, o_vmem):
            @pl.loop(0, _T * _NUM_CLASSES // 16)
            def _zero(k):
                counts[pl.ds(k * 16, 16)] = zeros

            @pl.loop(0, _N_CHUNKS)
            def _chunk(j):
                lj = l_vmem[pl.ds(j * 16, 16)]
                rj = r_vmem[pl.ds(j * 16, 16)]

                @pl.loop(0, _T)
                def _row(r):
                    xv = x_vmem[r, pl.ds(j * 16, 16)]
                    bins = jnp.where(xv > 0.5, rj, lj) + r * _NUM_CLASSES
                    plsc.addupdate_scatter(counts, [bins], ones)

            @pl.loop(0, _T)
            def _argmax(r):
                scores = []
                for c in range(4):
                    cnt = counts[pl.ds(r * _NUM_CLASSES + c * 16, 16)]
                    scores.append(
                        cnt * _NUM_CLASSES + (_NUM_CLASSES - 1 - c * 16) - lane
                    )
                m = jnp.maximum(
                    jnp.maximum(scores[0], scores[1]),
                    jnp.maximum(scores[2], scores[3]),
                )
                best = jnp.max(m)
                for c in range(4):
                    o_vmem[r, pl.ds(c * 16, 16)] = (scores[c] == best).astype(
                        jnp.int32
                    )

        blk_off = row_offset // _T
        pltpu.emit_pipeline(
            body,
            grid=(_N_WORKERS, n_tiles_per_worker),
            in_specs=[
                pl.BlockSpec(
                    (_T, _E), index_map=lambda i, j: (i * n_tiles_per_worker + j + blk_off, 0)
                )
            ],
            out_specs=[
                pl.BlockSpec(
                    (_T, _NUM_CLASSES),
                    index_map=lambda i, j: (i * n_tiles_per_worker + j, 0),
                )
            ],
            core_axis_name=("c", "s"),
            dimension_semantics=(pltpu.PARALLEL, pltpu.ARBITRARY),
        )(x_hbm, o_hbm)

    return sc_kernel(x, left, right)


def kernel(x, perms):
    left = perms[:, 0]
    right = perms[:, 1]
    parts = []
    if _SPLIT > 0:
        parts.append(_tc_part(x, perms, _SPLIT))
    if _SPLIT < _B:
        parts.append(_sc_part(x, left, right, _SPLIT, _B - _SPLIT))
    if len(parts) == 1:
        return parts[0]
    return jnp.concatenate(parts, axis=0)
